# baseline (speedup 1.0000x reference)
import functools

import jax
import jax.numpy as jnp
from jax import lax
from jax.experimental import pallas as pl
from jax.experimental.pallas import tpu as pltpu
from jax.experimental.pallas import tpu_sc as plsc

N = 10000
T = 3
E = 320000
THETA = 0.1
D0 = 128
D2 = 64

NSUB = 16
NCORE = 2
CH = 125
NCH = E // NSUB // CH
IB = 32
NIB = NCH // IB
KBUF = 2
NPAD = 10240
RPT = NPAD // NSUB
EW = E // (NCORE * NSUB)
NW = NCORE * NSUB

_coefs = [(1.0 - THETA) ** (T - t) for t in range(T)]
_csum = sum(_coefs)
_W = [c / _csum for c in _coefs]

_MESH = dict(core_axis_name="c", subcore_axis_name="s",
             num_cores=NCORE, num_subcores=NSUB)


def _make_spmm():
  scratch = [
      pltpu.VMEM_SHARED((NPAD, D0), jnp.float32),
      pltpu.VMEM((IB, CH), jnp.int32),
      pltpu.VMEM((IB, CH), jnp.int32),
      pltpu.VMEM((IB, CH), jnp.int32),
      pltpu.VMEM((IB, CH), jnp.int32),
  ] + [pltpu.VMEM((CH, D0), jnp.float32) for _ in range(KBUF)] \
    + [pltpu.SemaphoreType.DMA for _ in range(2 * KBUF)]

  @functools.partial(
      pl.kernel,
      out_type=jax.ShapeDtypeStruct((2 * T, NPAD, D0), jnp.float32),
      mesh=plsc.VectorSubcoreMesh(**_MESH),
      scratch_types=scratch,
  )
  def spmm(edge_hbm, x_hbm, p_hbm, accum, c0, r0, c1, r1, *rest):
    cols = [c0, c1]
    rows = [r0, r1]
    bufs = rest[:KBUF]
    gsems = rest[KBUF:2 * KBUF]
    ssems = rest[2 * KBUF:]
    c = lax.axis_index("c")
    s = lax.axis_index("s")

    for t in range(T):
      src = jnp.where(c == 0, t, T)
      out = c * T + t
      xs = x_hbm.at[src]

      def drain_g(b):
        pltpu.make_async_copy(xs.at[c0.at[0]], bufs[b], gsems[b]).wait()

      def drain_s(b):
        pltpu.make_async_copy(bufs[b], accum.at[r0.at[0]], ssems[b]).wait()

      pltpu.sync_copy(x_hbm.at[src, pl.ds(s * RPT, RPT)],
                      accum.at[pl.ds(s * RPT, RPT)])
      plsc.subcore_barrier()

      pltpu.sync_copy(edge_hbm.at[t, 1, s, pl.ds(0, IB)], cols[0])
      pltpu.sync_copy(edge_hbm.at[t, 0, s, pl.ds(0, IB)], rows[0])
      for b in range(KBUF - 1):
        pltpu.async_copy(xs.at[cols[0].at[b]], bufs[b], gsems[b])

      for nb in range(NIB):
        pb = nb % 2
        last_block = nb == NIB - 1
        if not last_block:
          pltpu.sync_copy(edge_hbm.at[t, 1, s, pl.ds((nb + 1) * IB, IB)],
                          cols[1 - pb])
          pltpu.sync_copy(edge_hbm.at[t, 0, s, pl.ds((nb + 1) * IB, IB)],
                          rows[1 - pb])

        def group(g, _, pb=pb, last_block=last_block, nb=nb):
          for bi in range(KBUF):
            jl = g * KBUF + bi
            drain_g(bi)
            pltpu.async_copy(bufs[bi], accum.at[rows[pb].at[jl]],
                             ssems[bi], add=True)
            bp = (bi + KBUF - 1) % KBUF
            if nb == 0 and bi == 0:
              @pl.when(g > 0)
              def _():
                drain_s(bp)
            else:
              drain_s(bp)
            jl3 = jl + KBUF - 1

            @pl.when(jl3 < IB)
            def _():
              pltpu.async_copy(xs.at[cols[pb].at[jl3]], bufs[bp], gsems[bp])

            if not last_block:
              @pl.when(jl3 >= IB)
              def _():
                pltpu.async_copy(xs.at[cols[1 - pb].at[jl3 - IB]],
                                 bufs[bp], gsems[bp])
          return 0

        lax.fori_loop(0, IB // KBUF, group, 0)

      drain_s((NCH - 1) % KBUF)
      plsc.subcore_barrier()
      pltpu.sync_copy(accum.at[pl.ds(s * RPT, RPT)],
                      p_hbm.at[out, pl.ds(s * RPT, RPT)])
      plsc.subcore_barrier()

  return spmm


def _make_hist():
  scratch = [
      pltpu.VMEM((T * NPAD,), jnp.float32),
      pltpu.VMEM((EW,), jnp.int32),
      pltpu.VMEM((16,), jnp.int32),
  ]

  @functools.partial(
      pl.kernel,
      out_type=jax.ShapeDtypeStruct((T, NCORE, NSUB, NPAD), jnp.float32),
      mesh=plsc.VectorSubcoreMesh(**_MESH),
      scratch_types=scratch,
      compiler_params=pltpu.CompilerParams(needs_layout_passes=False),
  )
  def hist(rows_hbm, cnt_hbm, hist_v, rows_v, ks_v):
    c = lax.axis_index("c")
    s = lax.axis_index("s")

    def zero(i, _):
      hist_v[pl.ds(i * 16, 16)] = jnp.zeros((16,), jnp.float32)
      return 0

    lax.fori_loop(0, T * NPAD // 16, zero, 0)
    for t in range(T):
      pltpu.sync_copy(rows_hbm.at[t, c, s], rows_v)
      ht = hist_v.at[pl.ds(t * NPAD, NPAD)]

      def count(i, _):
        idx = rows_v[pl.ds(i * 16, 16)]
        ks, _ = plsc.sort_key_val(idx, idx)
        pos = lax.iota(jnp.int32, 16)
        ks_v[...] = ks
        prev = plsc.load_gather(ks_v, [jnp.maximum(pos - 1, 0)])
        nxt = plsc.load_gather(ks_v, [jnp.minimum(pos + 1, 15)])
        is_first = (ks != prev) | (pos == 0)
        is_last = (ks != nxt) | (pos == 15)
        firstpos = plsc.cummax(jnp.where(is_first, pos, 0))
        runlen = (pos - firstpos + 1).astype(jnp.float32)
        plsc.addupdate_scatter(ht, [ks], runlen, mask=is_last)
        return 0

      lax.fori_loop(0, EW // 16, count, 0)
      pltpu.sync_copy(ht, cnt_hbm.at[t, c, s])

  return hist


_sc_cache = {}


def _sc(name, maker):
  if name not in _sc_cache:
    _sc_cache[name] = maker()
  return _sc_cache[name]


_BLK = 1280
_NB = NPAD // _BLK


def _layer_body(pf_ref, cn_ref, w_ref, h_ref):
  s = pl.program_id(1)
  jv = lax.broadcasted_iota(jnp.int32, (2 * T, 1, 1), 0)
  onehot = (jv == s).astype(jnp.float32)
  wgt = ((jv == T) * _W[0] + (jv == T + 1) * _W[1]
         + (jv == T + 2) * _W[2]).astype(jnp.float32)
  cw = jnp.where(s < T, onehot, wgt)
  comb = jnp.sum(cw * pf_ref[...], axis=0)
  tj = lax.broadcasted_iota(jnp.int32, (T * NW, 1), 0) // NW
  sel = (tj == s).astype(jnp.float32)
  twgt = ((tj == 0) * _W[0] + (tj == 1) * _W[1]
          + (tj == 2) * _W[2]).astype(jnp.float32)
  wv = jnp.where(s < T, sel, twgt)
  deg = 1.0 + jnp.sum(cn_ref[...] * wv, axis=0)
  pre = comb / deg[:, None]
  h = jnp.maximum(jnp.dot(pre, w_ref[...],
                          preferred_element_type=jnp.float32), 0.0)
  nrm = jnp.maximum(jnp.sqrt(jnp.sum(h * h, axis=1, keepdims=True)), 1e-12)
  h_ref[0] = h / nrm


def _tc_layer(p_feat, cnt, w):
  dout = w.shape[1]
  return pl.pallas_call(
      _layer_body,
      grid=(_NB, 4),
      in_specs=[
          pl.BlockSpec((2 * T, _BLK, D0), lambda b, s: (0, b, 0)),
          pl.BlockSpec((T * NW, _BLK), lambda b, s: (0, b)),
          pl.BlockSpec((D0, dout), lambda b, s: (0, 0)),
      ],
      out_specs=pl.BlockSpec((1, _BLK, dout), lambda b, s: (s, b, 0)),
      out_shape=jax.ShapeDtypeStruct((4, NPAD, dout), jnp.float32),
  )(p_feat, cnt, w)


def _attn_body(h_ref, wa_ref, o_ref):
  h = h_ref[...]
  g = h[T]
  q = jnp.dot(g, wa_ref[...], preferred_element_type=jnp.float32)
  inv_sqrt_d = 1.0 / (D2 ** 0.5)
  sc = [jnp.sum(h[t] * q, axis=1) * inv_sqrt_d for t in range(T)]
  m = jnp.maximum(jnp.maximum(sc[0], sc[1]), sc[2])
  ex = [jnp.exp(sc[t] - m) for t in range(T)]
  z = ex[0] + ex[1] + ex[2]
  agg = sum((ex[t] / z)[:, None] * h[t] for t in range(T))
  o_ref[:, :D2] = agg
  o_ref[:, D2:] = g


def _tc_attn(h2, wa):
  return pl.pallas_call(
      _attn_body,
      grid=(_NB,),
      in_specs=[
          pl.BlockSpec((4, _BLK, D2), lambda b: (0, b, 0)),
          pl.BlockSpec((D2, D2), lambda b: (0, 0)),
      ],
      out_specs=pl.BlockSpec((_BLK, 2 * D2), lambda b: (b, 0)),
      out_shape=jax.ShapeDtypeStruct((NPAD, 2 * D2), jnp.float32),
  )(h2, wa)


def kernel(edge_index, feat, W0, W1, Wa):
  edge_r = edge_index.reshape(T, 2, NSUB, NCH, CH)
  rows_r = edge_index[:, 0].reshape(T, NCORE, NSUB, EW)
  x1 = jnp.concatenate([feat, feat[T - 1:T]], axis=0)
  x1p = jnp.pad(x1, ((0, 0), (0, NPAD - N), (0, 0)))

  cnt = _sc("hist", _make_hist)(rows_r).reshape(T * NW, NPAD)
  p1 = _sc("spmm", _make_spmm)(edge_r, x1p)
  h1 = _tc_layer(p1, cnt, W0)
  p2 = _sc("spmm", _make_spmm)(edge_r, h1)
  h2 = _tc_layer(p2, cnt, W1)
  return _tc_attn(h2, Wa)[:N]


# speedup vs baseline: 1.1660x; 1.1660x over previous
import functools

import jax
import jax.numpy as jnp
from jax import lax
from jax.experimental import pallas as pl
from jax.experimental.pallas import tpu as pltpu
from jax.experimental.pallas import tpu_sc as plsc

N = 10000
T = 3
E = 320000
THETA = 0.1
D0 = 128
D2 = 64

NSUB = 16
NCORE = 2
CH = 50
NCH = E // NSUB // CH
IB = 40
NIB = NCH // IB
KBUF = 4
NPAD = 10240
RPT = NPAD // NSUB
EW = E // (NCORE * NSUB)
NW = NCORE * NSUB

_coefs = [(1.0 - THETA) ** (T - t) for t in range(T)]
_csum = sum(_coefs)
_W = [c / _csum for c in _coefs]

_MESH = dict(core_axis_name="c", subcore_axis_name="s",
             num_cores=NCORE, num_subcores=NSUB)


def _make_spmm():
  scratch = [
      pltpu.VMEM_SHARED((NPAD, D0), jnp.float32),
      pltpu.VMEM((IB, CH), jnp.int32),
      pltpu.VMEM((IB, CH), jnp.int32),
      pltpu.VMEM((IB, CH), jnp.int32),
      pltpu.VMEM((IB, CH), jnp.int32),
  ] + [pltpu.VMEM((CH, D0), jnp.float32) for _ in range(KBUF)] \
    + [pltpu.SemaphoreType.DMA for _ in range(2 * KBUF)]

  @functools.partial(
      pl.kernel,
      out_type=jax.ShapeDtypeStruct((2 * T, NPAD, D0), jnp.float32),
      mesh=plsc.VectorSubcoreMesh(**_MESH),
      scratch_types=scratch,
  )
  def spmm(edge_hbm, x_hbm, p_hbm, accum, c0, r0, c1, r1, *rest):
    cols = [c0, c1]
    rows = [r0, r1]
    bufs = rest[:KBUF]
    gsems = rest[KBUF:2 * KBUF]
    ssems = rest[2 * KBUF:]
    c = lax.axis_index("c")
    s = lax.axis_index("s")

    for t in range(T):
      src = jnp.where(c == 0, t, T)
      out = c * T + t
      xs = x_hbm.at[src]

      def drain_g(b):
        pltpu.make_async_copy(xs.at[c0.at[0]], bufs[b], gsems[b]).wait()

      def drain_s(b):
        pltpu.make_async_copy(bufs[b], accum.at[r0.at[0]], ssems[b]).wait()

      pltpu.sync_copy(x_hbm.at[src, pl.ds(s * RPT, RPT)],
                      accum.at[pl.ds(s * RPT, RPT)])
      plsc.subcore_barrier()

      pltpu.sync_copy(edge_hbm.at[t, 1, s, pl.ds(0, IB)], cols[0])
      pltpu.sync_copy(edge_hbm.at[t, 0, s, pl.ds(0, IB)], rows[0])
      for b in range(KBUF - 1):
        pltpu.async_copy(xs.at[cols[0].at[b]], bufs[b], gsems[b])

      for nb in range(NIB):
        pb = nb % 2
        last_block = nb == NIB - 1
        if not last_block:
          pltpu.sync_copy(edge_hbm.at[t, 1, s, pl.ds((nb + 1) * IB, IB)],
                          cols[1 - pb])
          pltpu.sync_copy(edge_hbm.at[t, 0, s, pl.ds((nb + 1) * IB, IB)],
                          rows[1 - pb])

        def group(g, _, pb=pb, last_block=last_block, nb=nb):
          for bi in range(KBUF):
            jl = g * KBUF + bi
            drain_g(bi)
            pltpu.async_copy(bufs[bi], accum.at[rows[pb].at[jl]],
                             ssems[bi], add=True)
            bp = (bi + KBUF - 1) % KBUF
            if nb == 0 and bi == 0:
              @pl.when(g > 0)
              def _():
                drain_s(bp)
            else:
              drain_s(bp)
            jl3 = jl + KBUF - 1

            @pl.when(jl3 < IB)
            def _():
              pltpu.async_copy(xs.at[cols[pb].at[jl3]], bufs[bp], gsems[bp])

            if not last_block:
              @pl.when(jl3 >= IB)
              def _():
                pltpu.async_copy(xs.at[cols[1 - pb].at[jl3 - IB]],
                                 bufs[bp], gsems[bp])
          return 0

        lax.fori_loop(0, IB // KBUF, group, 0)

      drain_s((NCH - 1) % KBUF)
      plsc.subcore_barrier()
      pltpu.sync_copy(accum.at[pl.ds(s * RPT, RPT)],
                      p_hbm.at[out, pl.ds(s * RPT, RPT)])
      plsc.subcore_barrier()

  return spmm


def _make_hist():
  scratch = [
      pltpu.VMEM((T * NPAD,), jnp.float32),
      pltpu.VMEM((EW,), jnp.int32),
      pltpu.VMEM((16,), jnp.int32),
  ]

  @functools.partial(
      pl.kernel,
      out_type=jax.ShapeDtypeStruct((T, NCORE, NSUB, NPAD), jnp.float32),
      mesh=plsc.VectorSubcoreMesh(**_MESH),
      scratch_types=scratch,
      compiler_params=pltpu.CompilerParams(needs_layout_passes=False),
  )
  def hist(rows_hbm, cnt_hbm, hist_v, rows_v, ks_v):
    c = lax.axis_index("c")
    s = lax.axis_index("s")

    def zero(i, _):
      hist_v[pl.ds(i * 16, 16)] = jnp.zeros((16,), jnp.float32)
      return 0

    lax.fori_loop(0, T * NPAD // 16, zero, 0)
    for t in range(T):
      pltpu.sync_copy(rows_hbm.at[t, c, s], rows_v)
      ht = hist_v.at[pl.ds(t * NPAD, NPAD)]

      def count(i, _):
        idx = rows_v[pl.ds(i * 16, 16)]
        ks, _ = plsc.sort_key_val(idx, idx)
        pos = lax.iota(jnp.int32, 16)
        ks_v[...] = ks
        prev = plsc.load_gather(ks_v, [jnp.maximum(pos - 1, 0)])
        nxt = plsc.load_gather(ks_v, [jnp.minimum(pos + 1, 15)])
        is_first = (ks != prev) | (pos == 0)
        is_last = (ks != nxt) | (pos == 15)
        firstpos = plsc.cummax(jnp.where(is_first, pos, 0))
        runlen = (pos - firstpos + 1).astype(jnp.float32)
        plsc.addupdate_scatter(ht, [ks], runlen, mask=is_last)
        return 0

      lax.fori_loop(0, EW // 16, count, 0)
      pltpu.sync_copy(ht, cnt_hbm.at[t, c, s])

  return hist


_sc_cache = {}


def _sc(name, maker):
  if name not in _sc_cache:
    _sc_cache[name] = maker()
  return _sc_cache[name]


_BLK = 1280
_NB = NPAD // _BLK


def _layer_body(pf_ref, cn_ref, w_ref, h_ref):
  s = pl.program_id(1)
  jv = lax.broadcasted_iota(jnp.int32, (2 * T, 1, 1), 0)
  onehot = (jv == s).astype(jnp.float32)
  wgt = ((jv == T) * _W[0] + (jv == T + 1) * _W[1]
         + (jv == T + 2) * _W[2]).astype(jnp.float32)
  cw = jnp.where(s < T, onehot, wgt)
  comb = jnp.sum(cw * pf_ref[...], axis=0)
  tj = lax.broadcasted_iota(jnp.int32, (T * NW, 1), 0) // NW
  sel = (tj == s).astype(jnp.float32)
  twgt = ((tj == 0) * _W[0] + (tj == 1) * _W[1]
          + (tj == 2) * _W[2]).astype(jnp.float32)
  wv = jnp.where(s < T, sel, twgt)
  deg = 1.0 + jnp.sum(cn_ref[...] * wv, axis=0)
  pre = comb / deg[:, None]
  h = jnp.maximum(jnp.dot(pre, w_ref[...],
                          preferred_element_type=jnp.float32), 0.0)
  nrm = jnp.maximum(jnp.sqrt(jnp.sum(h * h, axis=1, keepdims=True)), 1e-12)
  h_ref[0] = h / nrm


def _tc_layer(p_feat, cnt, w):
  dout = w.shape[1]
  return pl.pallas_call(
      _layer_body,
      grid=(_NB, 4),
      in_specs=[
          pl.BlockSpec((2 * T, _BLK, D0), lambda b, s: (0, b, 0)),
          pl.BlockSpec((T * NW, _BLK), lambda b, s: (0, b)),
          pl.BlockSpec((D0, dout), lambda b, s: (0, 0)),
      ],
      out_specs=pl.BlockSpec((1, _BLK, dout), lambda b, s: (s, b, 0)),
      out_shape=jax.ShapeDtypeStruct((4, NPAD, dout), jnp.float32),
  )(p_feat, cnt, w)


def _attn_body(h_ref, wa_ref, o_ref):
  h = h_ref[...]
  g = h[T]
  q = jnp.dot(g, wa_ref[...], preferred_element_type=jnp.float32)
  inv_sqrt_d = 1.0 / (D2 ** 0.5)
  sc = [jnp.sum(h[t] * q, axis=1) * inv_sqrt_d for t in range(T)]
  m = jnp.maximum(jnp.maximum(sc[0], sc[1]), sc[2])
  ex = [jnp.exp(sc[t] - m) for t in range(T)]
  z = ex[0] + ex[1] + ex[2]
  agg = sum((ex[t] / z)[:, None] * h[t] for t in range(T))
  o_ref[:, :D2] = agg
  o_ref[:, D2:] = g


def _tc_attn(h2, wa):
  return pl.pallas_call(
      _attn_body,
      grid=(_NB,),
      in_specs=[
          pl.BlockSpec((4, _BLK, D2), lambda b: (0, b, 0)),
          pl.BlockSpec((D2, D2), lambda b: (0, 0)),
      ],
      out_specs=pl.BlockSpec((_BLK, 2 * D2), lambda b: (b, 0)),
      out_shape=jax.ShapeDtypeStruct((NPAD, 2 * D2), jnp.float32),
  )(h2, wa)


def kernel(edge_index, feat, W0, W1, Wa):
  edge_r = edge_index.reshape(T, 2, NSUB, NCH, CH)
  rows_r = edge_index[:, 0].reshape(T, NCORE, NSUB, EW)
  x1 = jnp.concatenate([feat, feat[T - 1:T]], axis=0)
  x1p = jnp.pad(x1, ((0, 0), (0, NPAD - N), (0, 0)))

  cnt = _sc("hist", _make_hist)(rows_r).reshape(T * NW, NPAD)
  p1 = _sc("spmm", _make_spmm)(edge_r, x1p)
  h1 = _tc_layer(p1, cnt, W0)
  p2 = _sc("spmm", _make_spmm)(edge_r, h1)
  h2 = _tc_layer(p2, cnt, W1)
  return _tc_attn(h2, Wa)[:N]
